# baseline (speedup 1.0000x reference)
import functools

import jax
import jax.numpy as jnp
from jax import lax
from jax.experimental import pallas as pl
from jax.experimental.pallas import tpu as pltpu
from jax.experimental.pallas import tpu_sc as plsc

_EMBED_DIM = 128
_NUM_CORES = 2
_NUM_SUBCORES = 16
_NUM_WORKERS = _NUM_CORES * _NUM_SUBCORES
_CHUNK = 256


@functools.lru_cache(maxsize=None)
def _build(B, V, D, chunk):
    assert B % (_NUM_WORKERS * chunk) == 0
    b_per_w = B // _NUM_WORKERS
    nsteps = b_per_w // chunk
    assert nsteps >= 28 and (nsteps - 16) % 12 == 0
    ngroups = (nsteps - 16) // 12
    mesh = plsc.VectorSubcoreMesh(
        core_axis_name="c", subcore_axis_name="s",
        num_cores=_NUM_CORES, num_subcores=_NUM_SUBCORES)

    @functools.partial(
        pl.kernel,
        out_type=jax.ShapeDtypeStruct((B, D), jnp.float32),
        mesh=mesh,
        scratch_types=(
            [pltpu.VMEM_SHARED((V, D), jnp.float32)]
            + [pltpu.VMEM((chunk,), jnp.int32) for _ in range(4)]
            + [pltpu.VMEM((chunk, D), jnp.float32) for _ in range(3)]
            + [pltpu.SemaphoreType.DMA for _ in range(10)]
        ),
    )
    def launch(idx_hbm, table_hbm, out_hbm, tab_sh,
               iv0, iv1, iv2, iv3, rv0, rv1, rv2,
               is0, is1, is2, is3, gs0, gs1, gs2, ws0, ws1, ws2):
        idx_v = [iv0, iv1, iv2, iv3]
        rows_v = [rv0, rv1, rv2]
        isem = [is0, is1, is2, is3]
        gsem = [gs0, gs1, gs2]
        wsem = [ws0, ws1, ws2]
        cid = lax.axis_index("c")
        sid = lax.axis_index("s")
        wid = sid * _NUM_CORES + cid
        base = wid * b_per_w

        @pl.when(sid == 0)
        def _stage():
            pltpu.sync_copy(table_hbm, tab_sh)
        plsc.subcore_barrier()

        def start_i(g, s):
            pltpu.async_copy(idx_hbm.at[pl.ds(base + g * chunk, chunk)],
                             idx_v[s], isem[s])

        def wait_i(g, s):
            pltpu.make_async_copy(idx_hbm.at[pl.ds(base + g * chunk, chunk)],
                                  idx_v[s], isem[s]).wait()

        def start_g(s, b):
            pltpu.async_copy(tab_sh.at[idx_v[s]], rows_v[b], gsem[b])

        def wait_g(s, b):
            pltpu.make_async_copy(tab_sh.at[idx_v[s]], rows_v[b],
                                  gsem[b]).wait()

        def start_w(g, b):
            pltpu.async_copy(rows_v[b],
                             out_hbm.at[pl.ds(base + g * chunk, chunk)],
                             wsem[b])

        def wait_w(g, b):
            pltpu.make_async_copy(rows_v[b],
                                  out_hbm.at[pl.ds(base + g * chunk, chunk)],
                                  wsem[b]).wait()

        def body(g, jpar, do_wait_w=True, do_start_g=True, do_start_i=True):
            b = jpar % 3
            bn = (jpar + 1) % 3
            i0 = jpar % 4
            i1 = (jpar + 1) % 4
            i2 = (jpar + 2) % 4
            if do_wait_w:
                wait_w(g - 2, bn)
            if do_start_g:
                wait_i(g + 1, i1)
                start_g(i1, bn)
            if do_start_i:
                start_i(g + 2, i2)
            wait_g(i0, b)
            start_w(g, b)

        start_i(0, 0)
        start_i(1, 1)
        wait_i(0, 0)
        start_g(0, 0)
        body(0, 0, do_wait_w=False)
        body(1, 1, do_wait_w=False)
        body(2, 2)
        body(3, 3)

        @pl.loop(0, ngroups)
        def _group(gg):
            g0 = 4 + gg * 12
            for j in range(12):
                body(g0 + j, (4 + j) % 12)

        gE = nsteps - 12
        for j in range(12):
            g = gE + j
            body(g, g % 12,
                 do_start_g=(j < 11),
                 do_start_i=(j < 10))
        wait_w(nsteps - 2, (nsteps - 2) % 3)
        wait_w(nsteps - 1, (nsteps - 1) % 3)

    return launch


def kernel(idx, table):
    B = idx.shape[0] * idx.shape[1]
    idx_flat = idx.reshape(B)
    out = _build(B, table.shape[0], _EMBED_DIM, _CHUNK)(idx_flat, table)
    return out.reshape(idx.shape + (table.shape[1],))
